# baseline (device time: 124173 ns/iter reference)
import jax
import jax.numpy as jnp
from jax import lax
from jax.experimental import pallas as pl
from jax.experimental.pallas import tpu as pltpu

N = 16
M_PER = 512
D = 512


def kernel(partial, gamma):
    x = partial.reshape(N * M_PER, D)
    g = gamma.reshape(1, D)

    def body(x_ref, g_ref, out_ref, comm_ref, send_sems, recv_sems):
        my = lax.axis_index("i")
        left = lax.rem(my + N - 1, N)
        right = lax.rem(my + 1, N)

        barrier_sem = pltpu.get_barrier_semaphore()
        for nbr in (left, right):
            pl.semaphore_signal(
                barrier_sem, inc=1,
                device_id=(nbr,), device_id_type=pl.DeviceIdType.MESH,
            )
        pl.semaphore_wait(barrier_sem, 2)

        c0 = lax.rem(my + N - 1, N)
        comm_ref[0] = x_ref[pl.ds(c0 * M_PER, M_PER), :].astype(jnp.bfloat16)

        for s in range(N - 1):
            rdma = pltpu.make_async_remote_copy(
                src_ref=comm_ref.at[s],
                dst_ref=comm_ref.at[s + 1],
                send_sem=send_sems.at[s],
                recv_sem=recv_sems.at[s],
                device_id=(right,),
                device_id_type=pl.DeviceIdType.MESH,
            )
            rdma.start()
            rdma.wait()

            c = lax.rem(my + 2 * N - 2 - s, N)
            chunk = x_ref[pl.ds(c * M_PER, M_PER), :]
            if s < N - 2:
                comm_ref[s + 1] = (
                    comm_ref[s + 1].astype(jnp.float32) + chunk
                ).astype(jnp.bfloat16)
            else:
                acc = comm_ref[s + 1].astype(jnp.float32) + chunk
                rms = jnp.sqrt(
                    jnp.mean(acc * acc, axis=-1, keepdims=True) + 1e-6
                )
                out_ref[...] = acc / rms * g_ref[...]

    return pl.pallas_call(
        body,
        out_shape=jax.ShapeDtypeStruct((M_PER, D), jnp.float32),
        in_specs=[
            pl.BlockSpec(memory_space=pltpu.VMEM),
            pl.BlockSpec(memory_space=pltpu.VMEM),
        ],
        out_specs=pl.BlockSpec(memory_space=pltpu.VMEM),
        scratch_shapes=[
            pltpu.VMEM((N, M_PER, D), jnp.bfloat16),
            pltpu.SemaphoreType.DMA((N - 1,)),
            pltpu.SemaphoreType.DMA((N - 1,)),
        ],
        compiler_params=pltpu.CompilerParams(collective_id=0),
    )(x, g)


# device time: 101973 ns/iter; 1.2177x vs baseline; 1.2177x over previous
import jax
import jax.numpy as jnp
from jax import lax
from jax.experimental import pallas as pl
from jax.experimental.pallas import tpu as pltpu

N = 16
M_PER = 512
H = M_PER // 2
D = 512


def kernel(partial, gamma):
    x = partial.reshape(N * M_PER, D)
    g = gamma.reshape(1, D)

    def body(x_ref, g_ref, out_ref, cw_ref, ccw_ref, send_sems, recv_sems):
        my = lax.axis_index("i")
        left = lax.rem(my + N - 1, N)
        right = lax.rem(my + 1, N)

        barrier_sem = pltpu.get_barrier_semaphore()
        for nbr in (left, right):
            pl.semaphore_signal(
                barrier_sem, inc=1,
                device_id=(nbr,), device_id_type=pl.DeviceIdType.MESH,
            )
        pl.semaphore_wait(barrier_sem, 2)

        c_cw0 = lax.rem(my + N - 1, N)
        cw_ref[0] = x_ref[pl.ds(c_cw0 * M_PER, H), :].astype(jnp.bfloat16)
        c_ccw0 = lax.rem(my + 1, N)
        ccw_ref[0] = x_ref[pl.ds(c_ccw0 * M_PER + H, H), :].astype(jnp.bfloat16)

        sends = []
        for s in range(N - 1):
            rdma_cw = pltpu.make_async_remote_copy(
                src_ref=cw_ref.at[s],
                dst_ref=cw_ref.at[s + 1],
                send_sem=send_sems.at[s, 0],
                recv_sem=recv_sems.at[s, 0],
                device_id=(right,),
                device_id_type=pl.DeviceIdType.MESH,
            )
            rdma_ccw = pltpu.make_async_remote_copy(
                src_ref=ccw_ref.at[s],
                dst_ref=ccw_ref.at[s + 1],
                send_sem=send_sems.at[s, 1],
                recv_sem=recv_sems.at[s, 1],
                device_id=(left,),
                device_id_type=pl.DeviceIdType.MESH,
            )
            rdma_cw.start()
            rdma_ccw.start()
            sends.append(rdma_cw)
            sends.append(rdma_ccw)

            c_cw = lax.rem(my + 2 * N - 2 - s, N)
            c_ccw = lax.rem(my + 2 + s, N)

            rdma_cw.wait_recv()
            chunk_cw = x_ref[pl.ds(c_cw * M_PER, H), :]
            if s < N - 2:
                cw_ref[s + 1] = (
                    cw_ref[s + 1].astype(jnp.float32) + chunk_cw
                ).astype(jnp.bfloat16)
            else:
                acc = cw_ref[s + 1].astype(jnp.float32) + chunk_cw
                rms = jnp.sqrt(
                    jnp.mean(acc * acc, axis=-1, keepdims=True) + 1e-6
                )
                out_ref[pl.ds(0, H), :] = acc / rms * g_ref[...]

            rdma_ccw.wait_recv()
            chunk_ccw = x_ref[pl.ds(c_ccw * M_PER + H, H), :]
            if s < N - 2:
                ccw_ref[s + 1] = (
                    ccw_ref[s + 1].astype(jnp.float32) + chunk_ccw
                ).astype(jnp.bfloat16)
            else:
                acc = ccw_ref[s + 1].astype(jnp.float32) + chunk_ccw
                rms = jnp.sqrt(
                    jnp.mean(acc * acc, axis=-1, keepdims=True) + 1e-6
                )
                out_ref[pl.ds(H, H), :] = acc / rms * g_ref[...]

        for r in sends:
            r.wait_send()

    return pl.pallas_call(
        body,
        out_shape=jax.ShapeDtypeStruct((M_PER, D), jnp.float32),
        in_specs=[
            pl.BlockSpec(memory_space=pltpu.VMEM),
            pl.BlockSpec(memory_space=pltpu.VMEM),
        ],
        out_specs=pl.BlockSpec(memory_space=pltpu.VMEM),
        scratch_shapes=[
            pltpu.VMEM((N, H, D), jnp.bfloat16),
            pltpu.VMEM((N, H, D), jnp.bfloat16),
            pltpu.SemaphoreType.DMA((N - 1, 2)),
            pltpu.SemaphoreType.DMA((N - 1, 2)),
        ],
        compiler_params=pltpu.CompilerParams(collective_id=0),
    )(x, g)


# device time: 88498 ns/iter; 1.4031x vs baseline; 1.1523x over previous
import jax
import jax.numpy as jnp
from jax import lax
from jax.experimental import pallas as pl
from jax.experimental.pallas import tpu as pltpu

N = 16
M_PER = 512
H = M_PER // 2
D = 512

RING = (0, 4, 8, 12, 13, 9, 5, 1, 2, 6, 10, 14, 15, 11, 7, 3)
INV = tuple(RING.index(p) for p in range(N))


def kernel(partial, gamma):
    x = partial.reshape(N * M_PER, D)
    g = gamma.reshape(1, D)

    my = lax.axis_index("i")
    Rj = jnp.asarray(RING, dtype=jnp.int32)
    r = jnp.asarray(INV, dtype=jnp.int32)[my]
    right = Rj[(r + 1) % N]
    left = Rj[(r - 1) % N]
    send0_cw = Rj[(r - 1) % N]
    send0_ccw = Rj[(r + 1) % N]
    s_idx = jnp.arange(N - 1, dtype=jnp.int32)
    recv_cw = Rj[(r - 2 - s_idx) % N]
    recv_ccw = Rj[(r + 2 + s_idx) % N]
    meta = jnp.concatenate([
        jnp.stack([right, left, send0_cw, send0_ccw]),
        recv_cw,
        recv_ccw,
    ]).astype(jnp.int32)

    def body(meta_ref, x_ref, g_ref, out_ref,
             cw_ref, ccw_ref, send_sems, recv_sems):
        rt = meta_ref[0]
        lf = meta_ref[1]

        barrier_sem = pltpu.get_barrier_semaphore()
        for nbr in (lf, rt):
            pl.semaphore_signal(
                barrier_sem, inc=1,
                device_id=(nbr,), device_id_type=pl.DeviceIdType.MESH,
            )
        pl.semaphore_wait(barrier_sem, 2)

        cw_ref[0] = x_ref[pl.ds(meta_ref[2] * M_PER, H), :].astype(
            jnp.bfloat16)
        ccw_ref[0] = x_ref[pl.ds(meta_ref[3] * M_PER + H, H), :].astype(
            jnp.bfloat16)

        sends = []
        for s in range(N - 1):
            rdma_cw = pltpu.make_async_remote_copy(
                src_ref=cw_ref.at[s],
                dst_ref=cw_ref.at[s + 1],
                send_sem=send_sems.at[s, 0],
                recv_sem=recv_sems.at[s, 0],
                device_id=(rt,),
                device_id_type=pl.DeviceIdType.MESH,
            )
            rdma_ccw = pltpu.make_async_remote_copy(
                src_ref=ccw_ref.at[s],
                dst_ref=ccw_ref.at[s + 1],
                send_sem=send_sems.at[s, 1],
                recv_sem=recv_sems.at[s, 1],
                device_id=(lf,),
                device_id_type=pl.DeviceIdType.MESH,
            )
            rdma_cw.start()
            rdma_ccw.start()
            sends.append(rdma_cw)
            sends.append(rdma_ccw)

            c_cw = meta_ref[4 + s]
            c_ccw = meta_ref[4 + (N - 1) + s]

            rdma_cw.wait_recv()
            chunk_cw = x_ref[pl.ds(c_cw * M_PER, H), :]
            if s < N - 2:
                cw_ref[s + 1] = (
                    cw_ref[s + 1].astype(jnp.float32) + chunk_cw
                ).astype(jnp.bfloat16)
            else:
                acc = cw_ref[s + 1].astype(jnp.float32) + chunk_cw
                rms = jnp.sqrt(
                    jnp.mean(acc * acc, axis=-1, keepdims=True) + 1e-6
                )
                out_ref[pl.ds(0, H), :] = acc / rms * g_ref[...]

            rdma_ccw.wait_recv()
            chunk_ccw = x_ref[pl.ds(c_ccw * M_PER + H, H), :]
            if s < N - 2:
                ccw_ref[s + 1] = (
                    ccw_ref[s + 1].astype(jnp.float32) + chunk_ccw
                ).astype(jnp.bfloat16)
            else:
                acc = ccw_ref[s + 1].astype(jnp.float32) + chunk_ccw
                rms = jnp.sqrt(
                    jnp.mean(acc * acc, axis=-1, keepdims=True) + 1e-6
                )
                out_ref[pl.ds(H, H), :] = acc / rms * g_ref[...]

        for rr in sends:
            rr.wait_send()

    return pl.pallas_call(
        body,
        out_shape=jax.ShapeDtypeStruct((M_PER, D), jnp.float32),
        in_specs=[
            pl.BlockSpec(memory_space=pltpu.SMEM),
            pl.BlockSpec(memory_space=pltpu.VMEM),
            pl.BlockSpec(memory_space=pltpu.VMEM),
        ],
        out_specs=pl.BlockSpec(memory_space=pltpu.VMEM),
        scratch_shapes=[
            pltpu.VMEM((N, H, D), jnp.bfloat16),
            pltpu.VMEM((N, H, D), jnp.bfloat16),
            pltpu.SemaphoreType.DMA((N - 1, 2)),
            pltpu.SemaphoreType.DMA((N - 1, 2)),
        ],
        compiler_params=pltpu.CompilerParams(collective_id=0),
    )(meta, x, g)


# device time: 68328 ns/iter; 1.8173x vs baseline; 1.2952x over previous
import jax
import jax.numpy as jnp
from jax import lax
from jax.experimental import pallas as pl
from jax.experimental.pallas import tpu as pltpu

N = 16
M_PER = 512
H = M_PER // 2
SUB = 2
HS = H // SUB
D = 512

RING = (0, 4, 8, 12, 13, 9, 5, 1, 2, 6, 10, 14, 15, 11, 7, 3)
INV = tuple(RING.index(p) for p in range(N))


def kernel(partial, gamma):
    x = partial.reshape(N * M_PER, D)
    g = gamma.reshape(1, D)

    my = lax.axis_index("i")
    Rj = jnp.asarray(RING, dtype=jnp.int32)
    r = jnp.asarray(INV, dtype=jnp.int32)[my]
    right = Rj[(r + 1) % N]
    left = Rj[(r - 1) % N]
    send0_cw = Rj[(r - 1) % N]
    send0_ccw = Rj[(r + 1) % N]
    s_idx = jnp.arange(N - 1, dtype=jnp.int32)
    recv_cw = Rj[(r - 2 - s_idx) % N]
    recv_ccw = Rj[(r + 2 + s_idx) % N]
    meta = jnp.concatenate([
        jnp.stack([right, left, send0_cw, send0_ccw]),
        recv_cw,
        recv_ccw,
    ]).astype(jnp.int32)

    def body(meta_ref, x_ref, g_ref, out_ref,
             cw_ref, ccw_ref, send_sems, recv_sems):
        rt = meta_ref[0]
        lf = meta_ref[1]

        barrier_sem = pltpu.get_barrier_semaphore()
        for nbr in (lf, rt):
            pl.semaphore_signal(
                barrier_sem, inc=1,
                device_id=(nbr,), device_id_type=pl.DeviceIdType.MESH,
            )
        pl.semaphore_wait(barrier_sem, 2)

        def mk(s, k, dir_idx):
            ref = cw_ref if dir_idx == 0 else ccw_ref
            tgt = rt if dir_idx == 0 else lf
            return pltpu.make_async_remote_copy(
                src_ref=ref.at[s, k],
                dst_ref=ref.at[s + 1, k],
                send_sem=send_sems.at[s, dir_idx, k],
                recv_sem=recv_sems.at[s, dir_idx, k],
                device_id=(tgt,),
                device_id_type=pl.DeviceIdType.MESH,
            )

        sends = []

        for k in range(SUB):
            cw_ref[0, k] = x_ref[
                pl.ds(meta_ref[2] * M_PER + k * HS, HS), :
            ].astype(jnp.bfloat16)
            rd = mk(0, k, 0)
            rd.start()
            sends.append(rd)
            ccw_ref[0, k] = x_ref[
                pl.ds(meta_ref[3] * M_PER + H + k * HS, HS), :
            ].astype(jnp.bfloat16)
            rd = mk(0, k, 1)
            rd.start()
            sends.append(rd)

        for s in range(N - 1):
            c_cw = meta_ref[4 + s]
            c_ccw = meta_ref[4 + (N - 1) + s]
            for k in range(SUB):
                for dir_idx, ref, c, roff in (
                    (0, cw_ref, c_cw, 0),
                    (1, ccw_ref, c_ccw, H),
                ):
                    mk(s, k, dir_idx).wait_recv()
                    chunk = x_ref[pl.ds(c * M_PER + roff + k * HS, HS), :]
                    if s < N - 2:
                        ref[s + 1, k] = (
                            ref[s + 1, k].astype(jnp.float32) + chunk
                        ).astype(jnp.bfloat16)
                        rd = mk(s + 1, k, dir_idx)
                        rd.start()
                        sends.append(rd)
                    else:
                        acc = ref[s + 1, k].astype(jnp.float32) + chunk
                        rms = jnp.sqrt(
                            jnp.mean(acc * acc, axis=-1, keepdims=True)
                            + 1e-6
                        )
                        out_ref[pl.ds(roff + k * HS, HS), :] = (
                            acc / rms * g_ref[...]
                        )

        for rd in sends:
            rd.wait_send()

    return pl.pallas_call(
        body,
        out_shape=jax.ShapeDtypeStruct((M_PER, D), jnp.float32),
        in_specs=[
            pl.BlockSpec(memory_space=pltpu.SMEM),
            pl.BlockSpec(memory_space=pltpu.VMEM),
            pl.BlockSpec(memory_space=pltpu.VMEM),
        ],
        out_specs=pl.BlockSpec(memory_space=pltpu.VMEM),
        scratch_shapes=[
            pltpu.VMEM((N, SUB, HS, D), jnp.bfloat16),
            pltpu.VMEM((N, SUB, HS, D), jnp.bfloat16),
            pltpu.SemaphoreType.DMA((N - 1, 2, SUB)),
            pltpu.SemaphoreType.DMA((N - 1, 2, SUB)),
        ],
        compiler_params=pltpu.CompilerParams(collective_id=0),
    )(meta, x, g)


# device time: 64508 ns/iter; 1.9249x vs baseline; 1.0592x over previous
import jax
import jax.numpy as jnp
from jax import lax
from jax.experimental import pallas as pl
from jax.experimental.pallas import tpu as pltpu

N = 16
M_PER = 512
H = M_PER // 2
SUB = 4
HS = H // SUB
D = 512

RING = (0, 4, 8, 12, 13, 9, 5, 1, 2, 6, 10, 14, 15, 11, 7, 3)
INV = tuple(RING.index(p) for p in range(N))


def kernel(partial, gamma):
    x = partial.reshape(N * M_PER, D)
    g = gamma.reshape(1, D)

    my = lax.axis_index("i")
    Rj = jnp.asarray(RING, dtype=jnp.int32)
    r = jnp.asarray(INV, dtype=jnp.int32)[my]
    right = Rj[(r + 1) % N]
    left = Rj[(r - 1) % N]
    send0_cw = Rj[(r - 1) % N]
    send0_ccw = Rj[(r + 1) % N]
    s_idx = jnp.arange(N - 1, dtype=jnp.int32)
    recv_cw = Rj[(r - 2 - s_idx) % N]
    recv_ccw = Rj[(r + 2 + s_idx) % N]
    meta = jnp.concatenate([
        jnp.stack([right, left, send0_cw, send0_ccw]),
        recv_cw,
        recv_ccw,
    ]).astype(jnp.int32)

    def body(meta_ref, x_ref, g_ref, out_ref,
             cw_ref, ccw_ref, send_sems, recv_sems):
        rt = meta_ref[0]
        lf = meta_ref[1]

        barrier_sem = pltpu.get_barrier_semaphore()
        for nbr in (lf, rt):
            pl.semaphore_signal(
                barrier_sem, inc=1,
                device_id=(nbr,), device_id_type=pl.DeviceIdType.MESH,
            )
        pl.semaphore_wait(barrier_sem, 2)

        def mk(s, k, dir_idx):
            ref = cw_ref if dir_idx == 0 else ccw_ref
            tgt = rt if dir_idx == 0 else lf
            return pltpu.make_async_remote_copy(
                src_ref=ref.at[s, k],
                dst_ref=ref.at[s + 1, k],
                send_sem=send_sems.at[s, dir_idx, k],
                recv_sem=recv_sems.at[s, dir_idx, k],
                device_id=(tgt,),
                device_id_type=pl.DeviceIdType.MESH,
            )

        sends = []

        for k in range(SUB):
            cw_ref[0, k] = x_ref[
                pl.ds(meta_ref[2] * M_PER + k * HS, HS), :
            ].astype(jnp.bfloat16)
            rd = mk(0, k, 0)
            rd.start()
            sends.append(rd)
            ccw_ref[0, k] = x_ref[
                pl.ds(meta_ref[3] * M_PER + H + k * HS, HS), :
            ].astype(jnp.bfloat16)
            rd = mk(0, k, 1)
            rd.start()
            sends.append(rd)

        for s in range(N - 1):
            c_cw = meta_ref[4 + s]
            c_ccw = meta_ref[4 + (N - 1) + s]
            for k in range(SUB):
                for dir_idx, ref, c, roff in (
                    (0, cw_ref, c_cw, 0),
                    (1, ccw_ref, c_ccw, H),
                ):
                    mk(s, k, dir_idx).wait_recv()
                    chunk = x_ref[pl.ds(c * M_PER + roff + k * HS, HS), :]
                    if s < N - 2:
                        ref[s + 1, k] = (
                            ref[s + 1, k].astype(jnp.float32) + chunk
                        ).astype(jnp.bfloat16)
                        rd = mk(s + 1, k, dir_idx)
                        rd.start()
                        sends.append(rd)
                    else:
                        acc = ref[s + 1, k].astype(jnp.float32) + chunk
                        rms = jnp.sqrt(
                            jnp.mean(acc * acc, axis=-1, keepdims=True)
                            + 1e-6
                        )
                        out_ref[pl.ds(roff + k * HS, HS), :] = (
                            acc / rms * g_ref[...]
                        )

        for rd in sends:
            rd.wait_send()

    return pl.pallas_call(
        body,
        out_shape=jax.ShapeDtypeStruct((M_PER, D), jnp.float32),
        in_specs=[
            pl.BlockSpec(memory_space=pltpu.SMEM),
            pl.BlockSpec(memory_space=pltpu.VMEM),
            pl.BlockSpec(memory_space=pltpu.VMEM),
        ],
        out_specs=pl.BlockSpec(memory_space=pltpu.VMEM),
        scratch_shapes=[
            pltpu.VMEM((N, SUB, HS, D), jnp.bfloat16),
            pltpu.VMEM((N, SUB, HS, D), jnp.bfloat16),
            pltpu.SemaphoreType.DMA((N - 1, 2, SUB)),
            pltpu.SemaphoreType.DMA((N - 1, 2, SUB)),
        ],
        compiler_params=pltpu.CompilerParams(collective_id=0),
    )(meta, x, g)


# device time: 64217 ns/iter; 1.9336x vs baseline; 1.0045x over previous
import jax
import jax.numpy as jnp
from jax import lax
from jax.experimental import pallas as pl
from jax.experimental.pallas import tpu as pltpu

N = 16
M_PER = 512
H = M_PER // 2
SUB = 4
HS = H // SUB
D = 512

RING = (0, 4, 8, 12, 13, 9, 5, 1, 2, 6, 10, 14, 15, 11, 7, 3)
INV = tuple(RING.index(p) for p in range(N))


def kernel(partial, gamma):
    x = partial.reshape(N * M_PER, D)
    g = gamma.reshape(1, D)

    my = lax.axis_index("i")
    Rj = jnp.asarray(RING, dtype=jnp.int32)
    r = jnp.asarray(INV, dtype=jnp.int32)[my]
    right = Rj[(r + 1) % N]
    left = Rj[(r - 1) % N]
    send0_cw = Rj[(r - 1) % N]
    send0_ccw = Rj[(r + 1) % N]
    s_idx = jnp.arange(N - 1, dtype=jnp.int32)
    recv_cw = Rj[(r - 2 - s_idx) % N]
    recv_ccw = Rj[(r + 2 + s_idx) % N]
    meta = jnp.concatenate([
        jnp.stack([right, left, send0_cw, send0_ccw]),
        recv_cw,
        recv_ccw,
    ]).astype(jnp.int32)

    def body(meta_ref, x_ref, g_ref, out_ref,
             cw_ref, ccw_ref, xb_ref, send_sems, recv_sems):
        rt = meta_ref[0]
        lf = meta_ref[1]

        barrier_sem = pltpu.get_barrier_semaphore()
        for nbr in (lf, rt):
            pl.semaphore_signal(
                barrier_sem, inc=1,
                device_id=(nbr,), device_id_type=pl.DeviceIdType.MESH,
            )
        pl.semaphore_wait(barrier_sem, 2)

        def mk(s, k, dir_idx):
            ref = cw_ref if dir_idx == 0 else ccw_ref
            tgt = rt if dir_idx == 0 else lf
            return pltpu.make_async_remote_copy(
                src_ref=ref.at[s, k],
                dst_ref=ref.at[s + 1, k],
                send_sem=send_sems.at[s, dir_idx, k],
                recv_sem=recv_sems.at[s, dir_idx, k],
                device_id=(tgt,),
                device_id_type=pl.DeviceIdType.MESH,
            )

        sends = []

        for k in range(SUB):
            cw_ref[0, k] = x_ref[
                pl.ds(meta_ref[2] * M_PER + k * HS, HS), :
            ].astype(jnp.bfloat16)
            rd = mk(0, k, 0)
            rd.start()
            sends.append(rd)
            ccw_ref[0, k] = x_ref[
                pl.ds(meta_ref[3] * M_PER + H + k * HS, HS), :
            ].astype(jnp.bfloat16)
            rd = mk(0, k, 1)
            rd.start()
            sends.append(rd)

        xb_ref[...] = x_ref[...].astype(jnp.bfloat16)

        for s in range(N - 1):
            c_cw = meta_ref[4 + s]
            c_ccw = meta_ref[4 + (N - 1) + s]
            for k in range(SUB):
                for dir_idx, ref, c, roff in (
                    (0, cw_ref, c_cw, 0),
                    (1, ccw_ref, c_ccw, H),
                ):
                    mk(s, k, dir_idx).wait_recv()
                    if s < N - 2:
                        chunk = xb_ref[
                            pl.ds(c * M_PER + roff + k * HS, HS), :
                        ]
                        ref[s + 1, k] = ref[s + 1, k] + chunk
                        rd = mk(s + 1, k, dir_idx)
                        rd.start()
                        sends.append(rd)
                    else:
                        chunk = x_ref[
                            pl.ds(c * M_PER + roff + k * HS, HS), :
                        ]
                        acc = ref[s + 1, k].astype(jnp.float32) + chunk
                        rms = jnp.sqrt(
                            jnp.mean(acc * acc, axis=-1, keepdims=True)
                            + 1e-6
                        )
                        out_ref[pl.ds(roff + k * HS, HS), :] = (
                            acc / rms * g_ref[...]
                        )

        for rd in sends:
            rd.wait_send()

    return pl.pallas_call(
        body,
        out_shape=jax.ShapeDtypeStruct((M_PER, D), jnp.float32),
        in_specs=[
            pl.BlockSpec(memory_space=pltpu.SMEM),
            pl.BlockSpec(memory_space=pltpu.VMEM),
            pl.BlockSpec(memory_space=pltpu.VMEM),
        ],
        out_specs=pl.BlockSpec(memory_space=pltpu.VMEM),
        scratch_shapes=[
            pltpu.VMEM((N, SUB, HS, D), jnp.bfloat16),
            pltpu.VMEM((N, SUB, HS, D), jnp.bfloat16),
            pltpu.VMEM((N * M_PER, D), jnp.bfloat16),
            pltpu.SemaphoreType.DMA((N - 1, 2, SUB)),
            pltpu.SemaphoreType.DMA((N - 1, 2, SUB)),
        ],
        compiler_params=pltpu.CompilerParams(collective_id=0),
    )(meta, x, g)


# device time: 62283 ns/iter; 1.9937x vs baseline; 1.0311x over previous
import jax
import jax.numpy as jnp
from jax import lax
from jax.experimental import pallas as pl
from jax.experimental.pallas import tpu as pltpu

N = 16
M_PER = 512
SUB = 4
HS = M_PER // SUB
D = 512
R_CW = 8

RING = (0, 4, 8, 12, 13, 9, 5, 1, 2, 6, 10, 14, 15, 11, 7, 3)
INV = tuple(RING.index(p) for p in range(N))


def kernel(partial, gamma):
    x = partial.reshape(N * M_PER, D)
    g = gamma.reshape(1, D)

    my = lax.axis_index("i")
    Rj = jnp.asarray(RING, dtype=jnp.int32)
    r = jnp.asarray(INV, dtype=jnp.int32)[my]
    right = Rj[(r + 1) % N]
    left = Rj[(r - 1) % N]
    rounds = jnp.arange(R_CW + 1, dtype=jnp.int32)
    d_cw = Rj[(r + 8 - rounds) % N]
    d_ccw = Rj[(r + 8 + rounds) % N]
    meta = jnp.concatenate([
        jnp.stack([right, left]),
        d_cw,
        d_ccw,
    ]).astype(jnp.int32)

    def body(meta_ref, x_ref, g_ref, out_ref,
             cw_ref, ccw_ref, icw_ref, iccw_ref, xb_ref,
             send_sems, recv_sems):
        rt = meta_ref[0]
        lf = meta_ref[1]

        barrier_sem = pltpu.get_barrier_semaphore()
        for nbr in (lf, rt):
            pl.semaphore_signal(
                barrier_sem, inc=1,
                device_id=(nbr,), device_id_type=pl.DeviceIdType.MESH,
            )
        pl.semaphore_wait(barrier_sem, 2)

        def mk(s, dir_idx, k):
            ref = cw_ref if dir_idx == 0 else ccw_ref
            tgt = rt if dir_idx == 0 else lf
            if s == 0:
                iref = icw_ref if dir_idx == 0 else iccw_ref
                src = iref.at[k % 2]
            else:
                src = ref.at[s - 1, k]
            return pltpu.make_async_remote_copy(
                src_ref=src,
                dst_ref=ref.at[s, k],
                send_sem=send_sems.at[s, dir_idx, k],
                recv_sem=recv_sems.at[s, dir_idx, k],
                device_id=(tgt,),
                device_id_type=pl.DeviceIdType.MESH,
            )

        sends = []

        def start(s, dir_idx, k):
            rd = mk(s, dir_idx, k)
            rd.start()
            sends.append(rd)

        d0 = meta_ref[2]
        for k in (0, 2, 1, 3):
            dir_idx = 0 if k < 2 else 1
            iref = icw_ref if dir_idx == 0 else iccw_ref
            iref[k % 2] = x_ref[pl.ds(d0 * M_PER + k * HS, HS), :].astype(
                jnp.bfloat16)
            start(0, dir_idx, k)

        xb_ref[...] = x_ref[...].astype(jnp.bfloat16)

        for s in range(1, R_CW):
            dc = meta_ref[2 + s]
            dcc = meta_ref[11 + s]
            for k in range(SUB):
                for dir_idx, ref, c in ((0, cw_ref, dc), (1, ccw_ref, dcc)):
                    init_ks = (0, 1) if dir_idx == 0 else (2, 3)
                    chunk = xb_ref[pl.ds(c * M_PER + k * HS, HS), :]
                    if s == 1 and k not in init_ks:
                        ref[s - 1, k] = chunk
                    else:
                        mk(s - 1, dir_idx, k).wait_recv()
                        ref[s - 1, k] = ref[s - 1, k] + chunk
                    start(s, dir_idx, k)

        dmy = meta_ref[10]
        for k in range(SUB):
            mk(R_CW - 1, 0, k).wait_recv()
            mk(R_CW - 1, 1, k).wait_recv()
            acc = (
                cw_ref[R_CW - 1, k].astype(jnp.float32)
                + ccw_ref[R_CW - 1, k].astype(jnp.float32)
                + x_ref[pl.ds(dmy * M_PER + k * HS, HS), :]
            )
            rms = jnp.sqrt(
                jnp.mean(acc * acc, axis=-1, keepdims=True) + 1e-6
            )
            out_ref[pl.ds(k * HS, HS), :] = acc / rms * g_ref[...]

        for rd in sends:
            rd.wait_send()

    return pl.pallas_call(
        body,
        out_shape=jax.ShapeDtypeStruct((M_PER, D), jnp.float32),
        in_specs=[
            pl.BlockSpec(memory_space=pltpu.SMEM),
            pl.BlockSpec(memory_space=pltpu.VMEM),
            pl.BlockSpec(memory_space=pltpu.VMEM),
        ],
        out_specs=pl.BlockSpec(memory_space=pltpu.VMEM),
        scratch_shapes=[
            pltpu.VMEM((R_CW, SUB, HS, D), jnp.bfloat16),
            pltpu.VMEM((R_CW, SUB, HS, D), jnp.bfloat16),
            pltpu.VMEM((2, HS, D), jnp.bfloat16),
            pltpu.VMEM((2, HS, D), jnp.bfloat16),
            pltpu.VMEM((N * M_PER, D), jnp.bfloat16),
            pltpu.SemaphoreType.DMA((R_CW, 2, SUB)),
            pltpu.SemaphoreType.DMA((R_CW, 2, SUB)),
        ],
        compiler_params=pltpu.CompilerParams(collective_id=0),
    )(meta, x, g)


# device time: 62267 ns/iter; 1.9942x vs baseline; 1.0003x over previous
import jax
import jax.numpy as jnp
from jax import lax
from jax.experimental import pallas as pl
from jax.experimental.pallas import tpu as pltpu

N = 16
M_PER = 512
SUB = 4
HS = M_PER // SUB
D = 512
R_CW = 8

RING = (0, 4, 8, 12, 13, 9, 5, 1, 2, 6, 10, 14, 15, 11, 7, 3)
INV = tuple(RING.index(p) for p in range(N))


def kernel(partial, gamma):
    x = partial.reshape(N * M_PER, D)
    g = gamma.reshape(1, D)

    my = lax.axis_index("i")
    Rj = jnp.asarray(RING, dtype=jnp.int32)
    r = jnp.asarray(INV, dtype=jnp.int32)[my]
    right = Rj[(r + 1) % N]
    left = Rj[(r - 1) % N]
    rounds = jnp.arange(R_CW + 1, dtype=jnp.int32)
    d_cw = Rj[(r + 8 - rounds) % N]
    d_ccw = Rj[(r + 8 + rounds) % N]
    meta = jnp.concatenate([
        jnp.stack([right, left]),
        d_cw,
        d_ccw,
    ]).astype(jnp.int32)

    def body(meta_ref, x_ref, g_ref, out_ref,
             cw_ref, ccw_ref, icw_ref, iccw_ref, xb_ref,
             send_sems, recv_sems):
        rt = meta_ref[0]
        lf = meta_ref[1]

        barrier_sem = pltpu.get_barrier_semaphore()
        for nbr in (lf, rt):
            pl.semaphore_signal(
                barrier_sem, inc=1,
                device_id=(nbr,), device_id_type=pl.DeviceIdType.MESH,
            )
        pl.semaphore_wait(barrier_sem, 2)

        def mk(s, dir_idx, k):
            ref = cw_ref if dir_idx == 0 else ccw_ref
            tgt = rt if dir_idx == 0 else lf
            if s == 0:
                iref = icw_ref if dir_idx == 0 else iccw_ref
                src = iref.at[k % 2]
            else:
                src = ref.at[s - 1, k]
            return pltpu.make_async_remote_copy(
                src_ref=src,
                dst_ref=ref.at[s, k],
                send_sem=send_sems.at[s, dir_idx, k],
                recv_sem=recv_sems.at[s, dir_idx, k],
                device_id=(tgt,),
                device_id_type=pl.DeviceIdType.MESH,
            )

        sends = []

        def start(s, dir_idx, k):
            rd = mk(s, dir_idx, k)
            rd.start()
            sends.append(rd)

        d0 = meta_ref[2]
        for k in (0, 2, 1, 3):
            dir_idx = 0 if k < 2 else 1
            iref = icw_ref if dir_idx == 0 else iccw_ref
            iref[k % 2] = x_ref[pl.ds(d0 * M_PER + k * HS, HS), :].astype(
                jnp.bfloat16)
            start(0, dir_idx, k)

        xb_ref[...] = x_ref[...].astype(jnp.bfloat16)

        d1c = meta_ref[3]
        d1cc = meta_ref[12]
        for dir_idx, ref, c, ks in (
            (0, cw_ref, d1c, (2, 3)),
            (1, ccw_ref, d1cc, (0, 1)),
        ):
            for k in ks:
                ref[0, k] = xb_ref[pl.ds(c * M_PER + k * HS, HS), :]
                start(1, dir_idx, k)

        for s in range(1, R_CW):
            dc = meta_ref[2 + s]
            dcc = meta_ref[11 + s]
            for k in range(SUB):
                for dir_idx, ref, c in ((0, cw_ref, dc), (1, ccw_ref, dcc)):
                    init_ks = (0, 1) if dir_idx == 0 else (2, 3)
                    if s == 1 and k not in init_ks:
                        continue
                    chunk = xb_ref[pl.ds(c * M_PER + k * HS, HS), :]
                    mk(s - 1, dir_idx, k).wait_recv()
                    ref[s - 1, k] = ref[s - 1, k] + chunk
                    start(s, dir_idx, k)

        dmy = meta_ref[10]
        for k in range(SUB):
            mk(R_CW - 1, 0, k).wait_recv()
            mk(R_CW - 1, 1, k).wait_recv()
            acc = (
                cw_ref[R_CW - 1, k].astype(jnp.float32)
                + ccw_ref[R_CW - 1, k].astype(jnp.float32)
                + x_ref[pl.ds(dmy * M_PER + k * HS, HS), :]
            )
            rms = jnp.sqrt(
                jnp.mean(acc * acc, axis=-1, keepdims=True) + 1e-6
            )
            out_ref[pl.ds(k * HS, HS), :] = acc / rms * g_ref[...]

        for rd in sends:
            rd.wait_send()

    return pl.pallas_call(
        body,
        out_shape=jax.ShapeDtypeStruct((M_PER, D), jnp.float32),
        in_specs=[
            pl.BlockSpec(memory_space=pltpu.SMEM),
            pl.BlockSpec(memory_space=pltpu.VMEM),
            pl.BlockSpec(memory_space=pltpu.VMEM),
        ],
        out_specs=pl.BlockSpec(memory_space=pltpu.VMEM),
        scratch_shapes=[
            pltpu.VMEM((R_CW, SUB, HS, D), jnp.bfloat16),
            pltpu.VMEM((R_CW, SUB, HS, D), jnp.bfloat16),
            pltpu.VMEM((2, HS, D), jnp.bfloat16),
            pltpu.VMEM((2, HS, D), jnp.bfloat16),
            pltpu.VMEM((N * M_PER, D), jnp.bfloat16),
            pltpu.SemaphoreType.DMA((R_CW, 2, SUB)),
            pltpu.SemaphoreType.DMA((R_CW, 2, SUB)),
        ],
        compiler_params=pltpu.CompilerParams(collective_id=0),
    )(meta, x, g)
